# baseline (speedup 1.0000x reference)
import functools

import jax
import jax.numpy as jnp
from jax import lax
from jax.experimental import pallas as pl
from jax.experimental.pallas import tpu as pltpu
from jax.experimental.pallas import tpu_sc as plsc

B, T, D = 4, 2048, 2048
E, C = 8, 512
OUT = 16384
O_E = OUT // E
D_PK = D // 2
D_H = D // 2

N_HALF = 2
E_H = E // N_HALF
N_ROWS_H = E_H * B * C

NC, NS = 2, 16
NW = NC * NS
ROWS_PER_W = N_ROWS_H // NW
CHUNK = 32
N_CHUNKS = ROWS_PER_W // CHUNK


def _sc_gather_half(x_pk, idx_h):
    mesh = plsc.VectorSubcoreMesh(core_axis_name="c", subcore_axis_name="s")

    @functools.partial(
        pl.kernel,
        mesh=mesh,
        out_type=jax.ShapeDtypeStruct((N_ROWS_H, D_PK), jnp.float32),
        scratch_types=[
            pltpu.VMEM((ROWS_PER_W,), jnp.int32),
            pltpu.VMEM((CHUNK, D_PK), jnp.float32),
            pltpu.VMEM((CHUNK, D_PK), jnp.float32),
            pltpu.SemaphoreType.DMA,
            pltpu.SemaphoreType.DMA,
            pltpu.SemaphoreType.DMA,
            pltpu.SemaphoreType.DMA,
        ],
    )
    def gather_kernel(x_hbm, idx_hbm, out_hbm, idx_v, buf_a, buf_b, ga, gb, wa, wb):
        wid = lax.axis_index("s") * NC + lax.axis_index("c")
        base = wid * ROWS_PER_W
        pltpu.sync_copy(idx_hbm.at[pl.ds(base, ROWS_PER_W)], idx_v)

        bufs = (buf_a, buf_b)
        gsems = (ga, gb)
        wsems = (wa, wb)

        def gather_chunk(c):
            cp = pltpu.make_async_copy(
                x_hbm.at[idx_v.at[pl.ds(c * CHUNK, CHUNK)]], bufs[c % 2],
                gsems[c % 2],
            )
            cp.start()
            return cp

        def write_chunk(c):
            cp = pltpu.make_async_copy(
                bufs[c % 2], out_hbm.at[pl.ds(base + c * CHUNK, CHUNK)],
                wsems[c % 2],
            )
            cp.start()
            return cp

        g = [None] * N_CHUNKS
        w = [None] * N_CHUNKS
        g[0] = gather_chunk(0)
        g[1] = gather_chunk(1)
        g[0].wait()
        w[0] = write_chunk(0)
        for c in range(2, N_CHUNKS):
            w[c - 2].wait()
            g[c] = gather_chunk(c)
            g[c - 1].wait()
            w[c - 1] = write_chunk(c - 1)
        g[N_CHUNKS - 1].wait()
        w[N_CHUNKS - 1] = write_chunk(N_CHUNKS - 1)
        w[N_CHUNKS - 2].wait()
        w[N_CHUNKS - 1].wait()

    return gather_kernel(x_pk, idx_h)


def _mm_half(h, g_h, We, be, carry):
    def mm_kernel(a_ref, w_ref, b_ref, *rest):
        o_ref = rest[-1]
        a_u = lax.bitcast_convert_type(a_ref[0], jnp.uint32)
        a_lo = lax.bitcast_convert_type(a_u << jnp.uint32(16), jnp.float32)
        a_hi = lax.bitcast_convert_type(
            a_u & jnp.uint32(0xFFFF0000), jnp.float32
        )
        w = w_ref[0]
        acc = lax.dot_general(
            a_lo, w[:, :D_H], (((1,), (1,)), ((), ())),
            preferred_element_type=jnp.float32,
        )
        acc += lax.dot_general(
            a_hi, w[:, D_H:], (((1,), (1,)), ((), ())),
            preferred_element_type=jnp.float32,
        )
        o_ref[0, 0] = acc + b_ref[0]

    in_specs = [
        pl.BlockSpec((1, C, D_PK), lambda e, b: (e * B + b, 0, 0)),
        pl.BlockSpec((1, O_E, D), lambda e, b: (h * E_H + e, 0, 0)),
        pl.BlockSpec((1, 1, O_E), lambda e, b: (h * E_H + e, 0, 0)),
    ]
    args = (g_h, We, be)
    aliases = {}
    if carry is not None:
        in_specs.append(pl.BlockSpec(memory_space=pl.ANY))
        args = args + (carry,)
        aliases = {3: 0}
    return pl.pallas_call(
        mm_kernel,
        grid=(E_H, B),
        in_specs=in_specs,
        out_specs=pl.BlockSpec(
            (1, 1, C, O_E), lambda e, b: (b, h * E_H + e, 0, 0)
        ),
        out_shape=jax.ShapeDtypeStruct((B, E, C, O_E), jnp.float32),
        input_output_aliases=aliases,
    )(*args)


def kernel(x, expert_indices, W, b):
    x_bf = x.reshape(B * T, D).astype(jnp.bfloat16)
    x_pk = lax.bitcast_convert_type(
        jnp.transpose(x_bf.reshape(B * T, 2, D_H), (0, 2, 1)), jnp.float32
    )
    idx_ebc = jnp.transpose(expert_indices, (1, 0, 2))
    flat_idx = (
        idx_ebc + (jnp.arange(B, dtype=jnp.int32) * T)[None, :, None]
    ).reshape(N_HALF, N_ROWS_H)
    We = W.reshape(E, O_E, D)
    be = b.reshape(E, 1, O_E)

    gathered = [
        _sc_gather_half(x_pk, flat_idx[h]).reshape(E_H * B, C, D_PK)
        for h in range(N_HALF)
    ]
    out = None
    for h in range(N_HALF):
        out = _mm_half(h, gathered[h], We, be, out)
    return out


# speedup vs baseline: 1.1149x; 1.1149x over previous
import functools

import jax
import jax.numpy as jnp
from jax import lax
from jax.experimental import pallas as pl
from jax.experimental.pallas import tpu as pltpu
from jax.experimental.pallas import tpu_sc as plsc

B, T, D = 4, 2048, 2048
E, C = 8, 512
OUT = 16384
O_E = OUT // E
D_H = D // 2
D_PK = D // 2

PARTS = ((0, 3), (3, 3), (6, 2))

NC, NS = 2, 16
NW = NC * NS
CHUNK = 32


def _sc_gather_part(x_pk, idx_p, n_rows):
    mesh = plsc.VectorSubcoreMesh(core_axis_name="c", subcore_axis_name="s")
    rows_per_w = n_rows // NW
    n_chunks = rows_per_w // CHUNK

    @functools.partial(
        pl.kernel,
        mesh=mesh,
        out_type=jax.ShapeDtypeStruct((n_rows, D_PK), jnp.float32),
        scratch_types=[
            pltpu.VMEM((rows_per_w,), jnp.int32),
            pltpu.VMEM((CHUNK, D_PK), jnp.float32),
            pltpu.VMEM((CHUNK, D_PK), jnp.float32),
            pltpu.SemaphoreType.DMA,
            pltpu.SemaphoreType.DMA,
            pltpu.SemaphoreType.DMA,
            pltpu.SemaphoreType.DMA,
        ],
    )
    def gather_kernel(x_hbm, idx_hbm, out_hbm, idx_v, buf_a, buf_b, ga, gb, wa, wb):
        wid = lax.axis_index("s") * NC + lax.axis_index("c")
        base = wid * rows_per_w
        pltpu.sync_copy(idx_hbm.at[pl.ds(base, rows_per_w)], idx_v)

        bufs = (buf_a, buf_b)
        gsems = (ga, gb)
        wsems = (wa, wb)

        def gather_chunk(c):
            cp = pltpu.make_async_copy(
                x_hbm.at[idx_v.at[pl.ds(c * CHUNK, CHUNK)]], bufs[c % 2],
                gsems[c % 2],
            )
            cp.start()
            return cp

        def write_chunk(c):
            cp = pltpu.make_async_copy(
                bufs[c % 2], out_hbm.at[pl.ds(base + c * CHUNK, CHUNK)],
                wsems[c % 2],
            )
            cp.start()
            return cp

        g = [None] * n_chunks
        w = [None] * n_chunks
        g[0] = gather_chunk(0)
        g[1] = gather_chunk(1)
        g[0].wait()
        w[0] = write_chunk(0)
        for c in range(2, n_chunks):
            w[c - 2].wait()
            g[c] = gather_chunk(c)
            g[c - 1].wait()
            w[c - 1] = write_chunk(c - 1)
        g[n_chunks - 1].wait()
        w[n_chunks - 1] = write_chunk(n_chunks - 1)
        w[n_chunks - 2].wait()
        w[n_chunks - 1].wait()

    return gather_kernel(x_pk, idx_p)


def _mm_part(e0, n_e, g_p, We, be, carry):
    def mm_kernel(a_ref, w_ref, b_ref, *rest):
        o_ref = rest[-1]
        a_u = lax.bitcast_convert_type(a_ref[0], jnp.uint32)
        a_lo = lax.bitcast_convert_type(a_u << jnp.uint32(16), jnp.float32)
        a_hi = lax.bitcast_convert_type(
            a_u & jnp.uint32(0xFFFF0000), jnp.float32
        )
        w = w_ref[0]
        acc = lax.dot_general(
            a_lo, w[:, :D_H], (((1,), (1,)), ((), ())),
            preferred_element_type=jnp.float32,
        )
        acc += lax.dot_general(
            a_hi, w[:, D_H:], (((1,), (1,)), ((), ())),
            preferred_element_type=jnp.float32,
        )
        o_ref[0, 0] = acc + b_ref[0]

    in_specs = [
        pl.BlockSpec((1, C, D_PK), lambda e, b: (e * B + b, 0, 0)),
        pl.BlockSpec((1, O_E, D), lambda e, b: (e0 + e, 0, 0)),
        pl.BlockSpec((1, 1, O_E), lambda e, b: (e0 + e, 0, 0)),
    ]
    args = (g_p, We, be)
    aliases = {}
    if carry is not None:
        in_specs.append(pl.BlockSpec(memory_space=pl.ANY))
        args = args + (carry,)
        aliases = {3: 0}
    return pl.pallas_call(
        mm_kernel,
        grid=(n_e, B),
        in_specs=in_specs,
        out_specs=pl.BlockSpec(
            (1, 1, C, O_E), lambda e, b: (b, e0 + e, 0, 0)
        ),
        out_shape=jax.ShapeDtypeStruct((B, E, C, O_E), jnp.float32),
        input_output_aliases=aliases,
    )(*args)


def kernel(x, expert_indices, W, b):
    x2 = x.reshape(B * T, 2, D_H)
    w_lo = lax.bitcast_convert_type(
        x2[:, 0, :].astype(jnp.bfloat16), jnp.uint16
    ).astype(jnp.uint32)
    w_hi = lax.bitcast_convert_type(
        x2[:, 1, :].astype(jnp.bfloat16), jnp.uint16
    ).astype(jnp.uint32)
    x_pk = lax.bitcast_convert_type(
        w_lo | (w_hi << jnp.uint32(16)), jnp.float32
    )

    idx_ebc = jnp.transpose(expert_indices, (1, 0, 2))
    flat_idx = (
        idx_ebc + (jnp.arange(B, dtype=jnp.int32) * T)[None, :, None]
    ).reshape(E * B * C)
    We = W.reshape(E, O_E, D)
    be = b.reshape(E, 1, O_E)

    gathered = [
        _sc_gather_part(
            x_pk,
            lax.slice(flat_idx, (e0 * B * C,), ((e0 + n_e) * B * C,)),
            n_e * B * C,
        ).reshape(n_e * B, C, D_PK)
        for e0, n_e in PARTS
    ]
    out = None
    for (e0, n_e), g_p in zip(PARTS, gathered):
        out = _mm_part(e0, n_e, g_p, We, be, out)
    return out


# speedup vs baseline: 1.1643x; 1.0443x over previous
import functools

import jax
import jax.numpy as jnp
from jax import lax
from jax.experimental import pallas as pl
from jax.experimental.pallas import tpu as pltpu
from jax.experimental.pallas import tpu_sc as plsc

B, T, D = 4, 2048, 2048
E, C = 8, 512
OUT = 16384
O_E = OUT // E
D_H = D // 2
D_PK = D // 2

PARTS = ((0, 3), (3, 3), (6, 2))

NC, NS = 2, 16
NW = NC * NS
CHUNK = 32


def _sc_gather_part(x_pk, idx_p, n_rows):
    mesh = plsc.VectorSubcoreMesh(core_axis_name="c", subcore_axis_name="s")
    rows_per_w = n_rows // NW
    n_chunks = rows_per_w // CHUNK

    @functools.partial(
        pl.kernel,
        mesh=mesh,
        out_type=jax.ShapeDtypeStruct((n_rows, D_PK), jnp.float32),
        scratch_types=[
            pltpu.VMEM((rows_per_w,), jnp.int32),
            pltpu.VMEM((CHUNK, D_PK), jnp.float32),
            pltpu.VMEM((CHUNK, D_PK), jnp.float32),
            pltpu.SemaphoreType.DMA,
            pltpu.SemaphoreType.DMA,
            pltpu.SemaphoreType.DMA,
            pltpu.SemaphoreType.DMA,
        ],
    )
    def gather_kernel(x_hbm, idx_hbm, out_hbm, idx_v, buf_a, buf_b, ga, gb, wa, wb):
        wid = lax.axis_index("s") * NC + lax.axis_index("c")
        base = wid * rows_per_w
        pltpu.sync_copy(idx_hbm.at[pl.ds(base, rows_per_w)], idx_v)

        bufs = (buf_a, buf_b)
        gsems = (ga, gb)
        wsems = (wa, wb)

        def gather_chunk(c):
            cp = pltpu.make_async_copy(
                x_hbm.at[idx_v.at[pl.ds(c * CHUNK, CHUNK)]], bufs[c % 2],
                gsems[c % 2],
            )
            cp.start()
            return cp

        def write_chunk(c):
            cp = pltpu.make_async_copy(
                bufs[c % 2], out_hbm.at[pl.ds(base + c * CHUNK, CHUNK)],
                wsems[c % 2],
            )
            cp.start()
            return cp

        g = [None] * n_chunks
        w = [None] * n_chunks
        g[0] = gather_chunk(0)
        g[1] = gather_chunk(1)
        g[0].wait()
        w[0] = write_chunk(0)
        for c in range(2, n_chunks):
            w[c - 2].wait()
            g[c] = gather_chunk(c)
            g[c - 1].wait()
            w[c - 1] = write_chunk(c - 1)
        g[n_chunks - 1].wait()
        w[n_chunks - 1] = write_chunk(n_chunks - 1)
        w[n_chunks - 2].wait()
        w[n_chunks - 1].wait()

    return gather_kernel(x_pk, idx_p)


def _mm_part(e0, n_e, g_p, We, be, carry):
    def mm_kernel(a_ref, w_ref, b_ref, *rest):
        o_ref = rest[-1]
        a_u = lax.bitcast_convert_type(a_ref[0], jnp.uint32)
        a_lo = lax.bitcast_convert_type(a_u << jnp.uint32(16), jnp.float32)
        a_hi = lax.bitcast_convert_type(
            a_u & jnp.uint32(0xFFFF0000), jnp.float32
        )
        w = w_ref[0]
        acc = lax.dot_general(
            a_lo, w[:, :D_H], (((1,), (1,)), ((), ())),
            preferred_element_type=jnp.float32,
        )
        acc += lax.dot_general(
            a_hi, w[:, D_H:], (((1,), (1,)), ((), ())),
            preferred_element_type=jnp.float32,
        )
        o_ref[0, 0] = acc + b_ref[0]

    in_specs = [
        pl.BlockSpec((1, C, D_PK), lambda e, b: (e * B + b, 0, 0)),
        pl.BlockSpec((1, O_E, D), lambda e, b: (e0 + e, 0, 0)),
        pl.BlockSpec((1, 1, O_E), lambda e, b: (e0 + e, 0, 0)),
    ]
    args = (g_p, We, be)
    aliases = {}
    if carry is not None:
        in_specs.append(pl.BlockSpec(memory_space=pl.ANY))
        args = args + (carry,)
        aliases = {3: 0}
    return pl.pallas_call(
        mm_kernel,
        grid=(n_e, B),
        in_specs=in_specs,
        out_specs=pl.BlockSpec(
            (1, 1, C, O_E), lambda e, b: (b, e0 + e, 0, 0)
        ),
        out_shape=jax.ShapeDtypeStruct((B, E, C, O_E), jnp.float32),
        input_output_aliases=aliases,
    )(*args)


def kernel(x, expert_indices, W, b):
    x2 = x.reshape(B * T, 2, D_H)
    u_lo = lax.bitcast_convert_type(x2[:, 0, :], jnp.uint32)
    u_hi = lax.bitcast_convert_type(x2[:, 1, :], jnp.uint32)
    one = jnp.uint32(1)
    s16 = jnp.uint32(16)
    rnd = jnp.uint32(0x7FFF)
    t_lo = u_lo + rnd + ((u_lo >> s16) & one)
    t_hi = u_hi + rnd + ((u_hi >> s16) & one)
    x_pk = lax.bitcast_convert_type(
        (t_hi & jnp.uint32(0xFFFF0000)) | (t_lo >> s16), jnp.float32
    )

    idx_ebc = jnp.transpose(expert_indices, (1, 0, 2))
    flat_idx = (
        idx_ebc + (jnp.arange(B, dtype=jnp.int32) * T)[None, :, None]
    ).reshape(E * B * C)
    We = W.reshape(E, O_E, D)
    be = b.reshape(E, 1, O_E)

    gathered = [
        _sc_gather_part(
            x_pk,
            lax.slice(flat_idx, (e0 * B * C,), ((e0 + n_e) * B * C,)),
            n_e * B * C,
        ).reshape(n_e * B, C, D_PK)
        for e0, n_e in PARTS
    ]
    out = None
    for (e0, n_e), g_p in zip(PARTS, gathered):
        out = _mm_part(e0, n_e, g_p, We, be, out)
    return out


# speedup vs baseline: 1.7851x; 1.5332x over previous
import functools

import jax
import jax.numpy as jnp
from jax import lax
from jax.experimental import pallas as pl
from jax.experimental.pallas import tpu as pltpu
from jax.experimental.pallas import tpu_sc as plsc

B, T, D = 4, 2048, 2048
E, C = 8, 512
OUT = 16384
O_E = OUT // E
D_H = D // 2
D_PK = D // 2

PARTS = ((0, 3), (3, 3), (6, 2))

NC, NS = 2, 16
NW = NC * NS
CHUNK = 32


PACK_RB = 512


def _tc_pack(x2d):
    def pack_kernel(x_ref, o_ref):
        u = lax.bitcast_convert_type(x_ref[...], jnp.uint32)
        u_lo = u[:, :D_H]
        u_hi = u[:, D_H:]
        one = jnp.uint32(1)
        s16 = jnp.uint32(16)
        rnd = jnp.uint32(0x7FFF)
        t_lo = u_lo + rnd + ((u_lo >> s16) & one)
        t_hi = u_hi + rnd + ((u_hi >> s16) & one)
        o_ref[...] = lax.bitcast_convert_type(
            (t_hi & jnp.uint32(0xFFFF0000)) | (t_lo >> s16), jnp.float32
        )

    return pl.pallas_call(
        pack_kernel,
        grid=(B * T // PACK_RB,),
        in_specs=[pl.BlockSpec((PACK_RB, D), lambda i: (i, 0))],
        out_specs=pl.BlockSpec((PACK_RB, D_PK), lambda i: (i, 0)),
        out_shape=jax.ShapeDtypeStruct((B * T, D_PK), jnp.float32),
    )(x2d)


def _sc_gather_part(x_pk, idx_p, n_rows):
    mesh = plsc.VectorSubcoreMesh(core_axis_name="c", subcore_axis_name="s")
    rows_per_w = n_rows // NW
    n_chunks = rows_per_w // CHUNK

    @functools.partial(
        pl.kernel,
        mesh=mesh,
        out_type=jax.ShapeDtypeStruct((n_rows, D_PK), jnp.float32),
        scratch_types=[
            pltpu.VMEM((rows_per_w,), jnp.int32),
            pltpu.VMEM((CHUNK, D_PK), jnp.float32),
            pltpu.VMEM((CHUNK, D_PK), jnp.float32),
            pltpu.SemaphoreType.DMA,
            pltpu.SemaphoreType.DMA,
            pltpu.SemaphoreType.DMA,
            pltpu.SemaphoreType.DMA,
        ],
    )
    def gather_kernel(x_hbm, idx_hbm, out_hbm, idx_v, buf_a, buf_b, ga, gb, wa, wb):
        wid = lax.axis_index("s") * NC + lax.axis_index("c")
        base = wid * rows_per_w
        pltpu.sync_copy(idx_hbm.at[pl.ds(base, rows_per_w)], idx_v)

        bufs = (buf_a, buf_b)
        gsems = (ga, gb)
        wsems = (wa, wb)

        def gather_chunk(c):
            cp = pltpu.make_async_copy(
                x_hbm.at[idx_v.at[pl.ds(c * CHUNK, CHUNK)]], bufs[c % 2],
                gsems[c % 2],
            )
            cp.start()
            return cp

        def write_chunk(c):
            cp = pltpu.make_async_copy(
                bufs[c % 2], out_hbm.at[pl.ds(base + c * CHUNK, CHUNK)],
                wsems[c % 2],
            )
            cp.start()
            return cp

        g = [None] * n_chunks
        w = [None] * n_chunks
        g[0] = gather_chunk(0)
        g[1] = gather_chunk(1)
        g[0].wait()
        w[0] = write_chunk(0)
        for c in range(2, n_chunks):
            w[c - 2].wait()
            g[c] = gather_chunk(c)
            g[c - 1].wait()
            w[c - 1] = write_chunk(c - 1)
        g[n_chunks - 1].wait()
        w[n_chunks - 1] = write_chunk(n_chunks - 1)
        w[n_chunks - 2].wait()
        w[n_chunks - 1].wait()

    return gather_kernel(x_pk, idx_p)


def _mm_part(e0, n_e, g_p, We, be, carry):
    def mm_kernel(a_ref, w_ref, b_ref, *rest):
        o_ref = rest[-1]
        a_u = lax.bitcast_convert_type(a_ref[0], jnp.uint32)
        a_lo = lax.bitcast_convert_type(a_u << jnp.uint32(16), jnp.float32)
        a_hi = lax.bitcast_convert_type(
            a_u & jnp.uint32(0xFFFF0000), jnp.float32
        )
        w = w_ref[0]
        acc = lax.dot_general(
            a_lo, w[:, :D_H], (((1,), (1,)), ((), ())),
            preferred_element_type=jnp.float32,
        )
        acc += lax.dot_general(
            a_hi, w[:, D_H:], (((1,), (1,)), ((), ())),
            preferred_element_type=jnp.float32,
        )
        o_ref[0, 0] = acc + b_ref[0]

    in_specs = [
        pl.BlockSpec((1, C, D_PK), lambda e, b: (e * B + b, 0, 0)),
        pl.BlockSpec((1, O_E, D), lambda e, b: (e0 + e, 0, 0)),
        pl.BlockSpec((1, 1, O_E), lambda e, b: (e0 + e, 0, 0)),
    ]
    args = (g_p, We, be)
    aliases = {}
    if carry is not None:
        in_specs.append(pl.BlockSpec(memory_space=pl.ANY))
        args = args + (carry,)
        aliases = {3: 0}
    return pl.pallas_call(
        mm_kernel,
        grid=(n_e, B),
        in_specs=in_specs,
        out_specs=pl.BlockSpec(
            (1, 1, C, O_E), lambda e, b: (b, e0 + e, 0, 0)
        ),
        out_shape=jax.ShapeDtypeStruct((B, E, C, O_E), jnp.float32),
        input_output_aliases=aliases,
    )(*args)


def kernel(x, expert_indices, W, b):
    x_pk = _tc_pack(x.reshape(B * T, D))

    idx_ebc = jnp.transpose(expert_indices, (1, 0, 2))
    flat_idx = (
        idx_ebc + (jnp.arange(B, dtype=jnp.int32) * T)[None, :, None]
    ).reshape(E * B * C)
    We = W.reshape(E, O_E, D)
    be = b.reshape(E, 1, O_E)

    gathered = [
        _sc_gather_part(
            x_pk,
            lax.slice(flat_idx, (e0 * B * C,), ((e0 + n_e) * B * C,)),
            n_e * B * C,
        ).reshape(n_e * B, C, D_PK)
        for e0, n_e in PARTS
    ]
    out = None
    for (e0, n_e), g_p in zip(PARTS, gathered):
        out = _mm_part(e0, n_e, g_p, We, be, out)
    return out
